# baseline (speedup 1.0000x reference)
import functools

import jax
import jax.numpy as jnp
from jax import lax
from jax.experimental import pallas as pl
from jax.experimental.pallas import tpu as pltpu
from jax.experimental.pallas import tpu_sc as plsc

_M0 = 0.3
_NB = 512
_D2MAX = 8.5
_BINW = _D2MAX / _NB
_SCALE = _NB / _D2MAX
_MAGIC = 0x5F3759DF


def _dtm_sc_body(rows_per, xx_h, xy_h, w_h, gx_h, gy_h, out_h,
                 xx_v, xy_v, w_v, gx_v, gy_v, hw, hs, out_v):
    N = 2304
    chunks = rows_per // 16
    cid = lax.axis_index("c")
    sid = lax.axis_index("s")
    wid = sid * 2 + cid
    b = wid // 4
    mbase = (wid % 4) * rows_per

    pltpu.sync_copy(xx_h.at[b], xx_v)
    pltpu.sync_copy(xy_h.at[b], xy_v)
    pltpu.sync_copy(w_h.at[b], w_v)
    pltpu.sync_copy(gx_h.at[pl.ds(mbase, rows_per)], gx_v)
    pltpu.sync_copy(gy_h.at[pl.ds(mbase, rows_per)], gy_v)

    zf = jnp.zeros((16,), jnp.float32)
    lane = lax.iota(jnp.int32, 16)

    acc = lax.fori_loop(
        0, N // 16, lambda i, a: a + w_v[pl.ds(i * 16, 16)], zf)
    tot = acc[0]
    for j in range(1, 16):
        tot = tot + acc[j]
    wb = _M0 * tot

    def chunk_body(mi, _):
        gxc = gx_v[pl.ds(mi * 16, 16)]
        gyc = gy_v[pl.ds(mi * 16, 16)]

        @plsc.parallel_loop(0, _NB, unroll=8)
        def zero(j):
            hw[pl.ds(j * 16, 16)] = zf
            hs[pl.ds(j * 16, 16)] = zf

        @plsc.parallel_loop(0, N // 16, unroll=4)
        def sweep(i):
            base = i * 16
            xxv = xx_v[pl.ds(base, 16)]
            xyv = xy_v[pl.ds(base, 16)]
            wv = w_v[pl.ds(base, 16)]
            for j in range(16):
                dx = gxc - jnp.full((16,), xxv[j])
                dy = gyc - jnp.full((16,), xyv[j])
                d2 = jnp.maximum(dx * dx + dy * dy, 1e-12)
                wnv = jnp.full((16,), wv[j])
                si = jnp.minimum((d2 * _SCALE).astype(jnp.int32),
                                 _NB - 1) * 16 + lane
                plsc.addupdate_scatter(hw, [si], wnv)
                plsc.addupdate_scatter(hs, [si], wnv * d2)

        def scan_cond(carry):
            blk, cum_w, cum_s, w_c, s_c, c2, crossed = carry
            ncross = plsc.all_reduce_population_count(crossed)
            return jnp.logical_and(blk < _NB // 8, ncross[0] < 16)

        def scan_body(carry):
            blk, cum_w, cum_s, w_c, s_c, c2, crossed = carry
            for jj in range(8):
                j = blk * 8 + jj
                cum_w = cum_w + hw[pl.ds(j * 16, 16)]
                cum_s = cum_s + hs[pl.ds(j * 16, 16)]
                hit = jnp.logical_and(jnp.logical_not(crossed), cum_w >= wb)
                w_c = jnp.where(hit, cum_w, w_c)
                s_c = jnp.where(hit, cum_s, s_c)
                c2 = jnp.where(hit, jnp.full((16,), j, jnp.int32), c2)
                crossed = jnp.logical_or(crossed, hit)
            return blk + 1, cum_w, cum_s, w_c, s_c, c2, crossed

        init = (jnp.int32(0), zf, zf, zf, zf,
                jnp.full((16,), _NB - 1, jnp.int32),
                jnp.zeros((16,), jnp.bool_))
        _, cum_w, cum_s, w_c, s_c, c2, crossed = lax.while_loop(
            scan_cond, scan_body, init)
        w_c = jnp.where(crossed, w_c, cum_w)
        s_c = jnp.where(crossed, s_c, cum_s)

        tau = (c2 + 1).astype(jnp.float32) * _BINW
        x = (s_c + tau * (wb - w_c)) / wb
        yi = _MAGIC - (plsc.bitcast(x, jnp.int32) >> 1)
        y = plsc.bitcast(yi, jnp.float32)
        y = y * (1.5 - 0.5 * x * y * y)
        y = y * (1.5 - 0.5 * x * y * y)
        y = y * (1.5 - 0.5 * x * y * y)
        out_v[pl.ds(mi * 16, 16)] = x * y
        return 0

    lax.fori_loop(0, chunks, chunk_body, 0)
    pltpu.sync_copy(out_v, out_h.at[pl.ds(wid * rows_per, rows_per)])


_ITERS = 16


def _dtm_tc_kernel(x_ref, w_ref, g_ref, o_ref):
    B = x_ref.shape[0]
    Mt = g_ref.shape[0]
    gx = g_ref[:, 0:1]
    gy = g_ref[:, 1:2]
    for b in range(B):
        xx = x_ref[b, 0:1, :]
        xy = x_ref[b, 1:2, :]
        w = w_ref[b]
        wb = _M0 * jnp.sum(w)
        dx = gx - xx
        dy = gy - xy
        d2 = jnp.maximum(dx * dx + dy * dy, 1e-12)
        wd2 = w * d2
        lo = jnp.zeros((Mt, 1), jnp.float32)
        hi = jnp.full((Mt, 1), 8.5, jnp.float32)

        def body(_, carry):
            lo, hi = carry
            mid = 0.5 * (lo + hi)
            wsum = jnp.sum(jnp.where(d2 < mid, w, 0.0), axis=1, keepdims=True)
            pred = wsum < wb
            return jnp.where(pred, mid, lo), jnp.where(pred, hi, mid)

        lo, hi = jax.lax.fori_loop(0, _ITERS, body, (lo, hi))
        tau = 0.5 * (lo + hi)
        mask = d2 < tau
        wl = jnp.sum(jnp.where(mask, w, 0.0), axis=1, keepdims=True)
        sl = jnp.sum(jnp.where(mask, wd2, 0.0), axis=1, keepdims=True)
        o_ref[:, b : b + 1] = jnp.sqrt((sl + tau * (wb - wl)) / wb)


_M_SC = 1536


def kernel(input, weight, grid):
    B, N, _ = input.shape
    M = grid.shape[0]
    xx = input[:, :, 0]
    xy = input[:, :, 1]
    gx = grid[:_M_SC, 0]
    gy = grid[:_M_SC, 1]
    mesh = plsc.VectorSubcoreMesh(core_axis_name="c", subcore_axis_name="s")
    rows_per = _M_SC // 4
    run = pl.kernel(
        functools.partial(_dtm_sc_body, rows_per),
        mesh=mesh,
        compiler_params=pltpu.CompilerParams(
            needs_layout_passes=False, use_tc_tiling_on_sc=False),
        out_type=jax.ShapeDtypeStruct((B * _M_SC,), jnp.float32),
        scratch_types=[
            pltpu.VMEM((N,), jnp.float32),
            pltpu.VMEM((N,), jnp.float32),
            pltpu.VMEM((N,), jnp.float32),
            pltpu.VMEM((rows_per,), jnp.float32),
            pltpu.VMEM((rows_per,), jnp.float32),
            pltpu.VMEM((_NB * 16,), jnp.float32),
            pltpu.VMEM((_NB * 16,), jnp.float32),
            pltpu.VMEM((rows_per,), jnp.float32),
        ],
    )
    out_sc = run(xx, xy, weight, gx, gy).reshape(B, _M_SC)

    m_tc = M - _M_SC
    mt = 128
    x_t = jnp.swapaxes(input, 1, 2)
    w3 = weight[:, None, :]
    out_tc = pl.pallas_call(
        _dtm_tc_kernel,
        grid=(m_tc // mt,),
        in_specs=[
            pl.BlockSpec((B, 2, N), lambda m: (0, 0, 0)),
            pl.BlockSpec((B, 1, N), lambda m: (0, 0, 0)),
            pl.BlockSpec((mt, 2), lambda m: (m, 0)),
        ],
        out_specs=pl.BlockSpec((mt, B), lambda m: (m, 0)),
        out_shape=jax.ShapeDtypeStruct((m_tc, B), jnp.float32),
    )(x_t, w3, grid[_M_SC:])
    return jnp.concatenate([out_sc, out_tc.T], axis=1)


# speedup vs baseline: 1.5313x; 1.5313x over previous
import functools

import jax
import jax.numpy as jnp
from jax import lax
from jax.experimental import pallas as pl
from jax.experimental.pallas import tpu as pltpu
from jax.experimental.pallas import tpu_sc as plsc

_M0 = 0.3
_NB = 384
_D2MAX = 8.5
_BINW = _D2MAX / _NB
_SCALE = _NB / _D2MAX
_MAGIC = 0x5F3759DF


def _dtm_sc_body(rows_per, xx_h, xy_h, w_h, gx_h, gy_h, out_h,
                 xx_v, xy_v, w_v, gx_v, gy_v, hw, hs, out_v):
    N = 2304
    chunks = rows_per // 16
    cid = lax.axis_index("c")
    sid = lax.axis_index("s")
    wid = sid * 2 + cid
    b = wid // 4
    mbase = (wid % 4) * rows_per

    pltpu.sync_copy(xx_h.at[b], xx_v)
    pltpu.sync_copy(xy_h.at[b], xy_v)
    pltpu.sync_copy(w_h.at[b], w_v)
    pltpu.sync_copy(gx_h.at[pl.ds(mbase, rows_per)], gx_v)
    pltpu.sync_copy(gy_h.at[pl.ds(mbase, rows_per)], gy_v)

    zf = jnp.zeros((16,), jnp.float32)
    lane = lax.iota(jnp.int32, 16)

    acc = lax.fori_loop(
        0, N // 16, lambda i, a: a + w_v[pl.ds(i * 16, 16)], zf)
    tot = acc[0]
    for j in range(1, 16):
        tot = tot + acc[j]
    wb = _M0 * tot

    def chunk_body(mi, _):
        gxc = gx_v[pl.ds(mi * 16, 16)]
        gyc = gy_v[pl.ds(mi * 16, 16)]

        @plsc.parallel_loop(0, _NB, unroll=8)
        def zero(j):
            hw[pl.ds(j * 16, 16)] = zf
            hs[pl.ds(j * 16, 16)] = zf

        @plsc.parallel_loop(0, N // 16, unroll=2)
        def sweep(i):
            base = i * 16
            xxv = xx_v[pl.ds(base, 16)]
            xyv = xy_v[pl.ds(base, 16)]
            wv = w_v[pl.ds(base, 16)]
            for j in range(16):
                dx = gxc - jnp.full((16,), xxv[j])
                dy = gyc - jnp.full((16,), xyv[j])
                d2 = jnp.maximum(dx * dx + dy * dy, 1e-12)
                wnv = jnp.full((16,), wv[j])
                si = jnp.minimum((d2 * _SCALE).astype(jnp.int32),
                                 _NB - 1) * 16 + lane
                plsc.addupdate_scatter(hw, [si], wnv)
                plsc.addupdate_scatter(hs, [si], wnv * d2)

        def scan_cond(carry):
            blk, cum_w, cum_s, w_c, s_c, c2, crossed = carry
            ncross = plsc.all_reduce_population_count(crossed)
            return jnp.logical_and(blk < _NB // 8, ncross[0] < 16)

        def scan_body(carry):
            blk, cum_w, cum_s, w_c, s_c, c2, crossed = carry
            for jj in range(8):
                j = blk * 8 + jj
                cum_w = cum_w + hw[pl.ds(j * 16, 16)]
                cum_s = cum_s + hs[pl.ds(j * 16, 16)]
                hit = jnp.logical_and(jnp.logical_not(crossed), cum_w >= wb)
                w_c = jnp.where(hit, cum_w, w_c)
                s_c = jnp.where(hit, cum_s, s_c)
                c2 = jnp.where(hit, jnp.full((16,), j, jnp.int32), c2)
                crossed = jnp.logical_or(crossed, hit)
            return blk + 1, cum_w, cum_s, w_c, s_c, c2, crossed

        init = (jnp.int32(0), zf, zf, zf, zf,
                jnp.full((16,), _NB - 1, jnp.int32),
                jnp.zeros((16,), jnp.bool_))
        _, cum_w, cum_s, w_c, s_c, c2, crossed = lax.while_loop(
            scan_cond, scan_body, init)
        w_c = jnp.where(crossed, w_c, cum_w)
        s_c = jnp.where(crossed, s_c, cum_s)

        tau = (c2 + 1).astype(jnp.float32) * _BINW
        x = (s_c + tau * (wb - w_c)) / wb
        yi = _MAGIC - (plsc.bitcast(x, jnp.int32) >> 1)
        y = plsc.bitcast(yi, jnp.float32)
        y = y * (1.5 - 0.5 * x * y * y)
        y = y * (1.5 - 0.5 * x * y * y)
        y = y * (1.5 - 0.5 * x * y * y)
        out_v[pl.ds(mi * 16, 16)] = x * y
        return 0

    lax.fori_loop(0, chunks, chunk_body, 0)
    pltpu.sync_copy(out_v, out_h.at[pl.ds(wid * rows_per, rows_per)])


_ITERS = 16


def _dtm_tc_kernel(x_ref, w_ref, g_ref, o_ref):
    B = x_ref.shape[0]
    Mt = g_ref.shape[0]
    gx = g_ref[:, 0:1]
    gy = g_ref[:, 1:2]
    for b in range(B):
        xx = x_ref[b, 0:1, :]
        xy = x_ref[b, 1:2, :]
        w = w_ref[b]
        wb = _M0 * jnp.sum(w)
        dx = gx - xx
        dy = gy - xy
        d2 = jnp.maximum(dx * dx + dy * dy, 1e-12)
        wd2 = w * d2
        lo = jnp.zeros((Mt, 1), jnp.float32)
        hi = jnp.full((Mt, 1), 8.5, jnp.float32)

        def body(_, carry):
            lo, hi = carry
            mid = 0.5 * (lo + hi)
            wsum = jnp.sum(jnp.where(d2 < mid, w, 0.0), axis=1, keepdims=True)
            pred = wsum < wb
            return jnp.where(pred, mid, lo), jnp.where(pred, hi, mid)

        lo, hi = jax.lax.fori_loop(0, _ITERS, body, (lo, hi))
        tau = 0.5 * (lo + hi)
        mask = d2 < tau
        wl = jnp.sum(jnp.where(mask, w, 0.0), axis=1, keepdims=True)
        sl = jnp.sum(jnp.where(mask, wd2, 0.0), axis=1, keepdims=True)
        o_ref[:, b : b + 1] = jnp.sqrt((sl + tau * (wb - wl)) / wb)


_M_SC = 1536


def kernel(input, weight, grid):
    B, N, _ = input.shape
    M = grid.shape[0]
    xx = input[:, :, 0]
    xy = input[:, :, 1]
    gx = grid[:_M_SC, 0]
    gy = grid[:_M_SC, 1]
    mesh = plsc.VectorSubcoreMesh(core_axis_name="c", subcore_axis_name="s")
    rows_per = _M_SC // 4
    run = pl.kernel(
        functools.partial(_dtm_sc_body, rows_per),
        mesh=mesh,
        compiler_params=pltpu.CompilerParams(
            needs_layout_passes=False, use_tc_tiling_on_sc=False),
        out_type=jax.ShapeDtypeStruct((B * _M_SC,), jnp.float32),
        scratch_types=[
            pltpu.VMEM((N,), jnp.float32),
            pltpu.VMEM((N,), jnp.float32),
            pltpu.VMEM((N,), jnp.float32),
            pltpu.VMEM((rows_per,), jnp.float32),
            pltpu.VMEM((rows_per,), jnp.float32),
            pltpu.VMEM((_NB * 16,), jnp.float32),
            pltpu.VMEM((_NB * 16,), jnp.float32),
            pltpu.VMEM((rows_per,), jnp.float32),
        ],
    )
    out_sc = run(xx, xy, weight, gx, gy).reshape(B, _M_SC)

    m_tc = M - _M_SC
    mt = 128
    x_t = jnp.swapaxes(input, 1, 2)
    w3 = weight[:, None, :]
    out_tc = pl.pallas_call(
        _dtm_tc_kernel,
        grid=(m_tc // mt,),
        in_specs=[
            pl.BlockSpec((B, 2, N), lambda m: (0, 0, 0)),
            pl.BlockSpec((B, 1, N), lambda m: (0, 0, 0)),
            pl.BlockSpec((mt, 2), lambda m: (m, 0)),
        ],
        out_specs=pl.BlockSpec((mt, B), lambda m: (m, 0)),
        out_shape=jax.ShapeDtypeStruct((m_tc, B), jnp.float32),
    )(x_t, w3, grid[_M_SC:])
    return jnp.concatenate([out_sc, out_tc.T], axis=1)
